# baseline (speedup 1.0000x reference)
import jax
import jax.numpy as jnp
from jax.experimental import pallas as pl

_N = 10000
_EPS = 1e-5
_NEG_SLOPE = 0.2


def _gat_conv(x, edge_index, W, a_src, a_dst, b):
    h = x @ W
    src = edge_index[0]
    dst = edge_index[1]
    loop = jnp.arange(_N, dtype=src.dtype)
    src = jnp.concatenate([src, loop])
    dst = jnp.concatenate([dst, loop])
    al_src = h @ a_src
    al_dst = h @ a_dst
    e = al_src[src] + al_dst[dst]
    e = jax.nn.leaky_relu(e, _NEG_SLOPE)
    e_max = jax.ops.segment_max(e, dst, num_segments=_N)
    e = jnp.exp(e - e_max[dst])
    denom = jax.ops.segment_sum(e, dst, num_segments=_N)
    alpha = e / denom[dst]
    out = jax.ops.segment_sum(alpha[:, None] * h[src], dst, num_segments=_N)
    return out + b


def _batch_norm(x, g, b):
    m = jnp.mean(x, axis=0)
    v = jnp.var(x, axis=0)
    return g * (x - m) / jnp.sqrt(v + _EPS) + b


def kernel(x, edge_index, W1, a_src1, a_dst1, b1, g1, be1, W2, a_src2, a_dst2, b2, g2, be2, fc_w, fc_b):
    h = _gat_conv(x, edge_index, W1, a_src1, a_dst1, b1)
    h = _batch_norm(h, g1, be1)
    h = jax.nn.relu(h)
    h = _gat_conv(h, edge_index, W2, a_src2, a_dst2, b2)
    h = _batch_norm(h, g2, be2)
    pooled = jnp.mean(h, axis=0, keepdims=True)
    out = pooled @ fc_w + fc_b
    return out.reshape(-1)


# speedup vs baseline: 1.1682x; 1.1682x over previous
import functools

import jax
import jax.numpy as jnp
from jax import lax
from jax.experimental import pallas as pl
from jax.experimental.pallas import tpu as pltpu
from jax.experimental.pallas import tpu_sc as plsc

_N = 10000
_EPS = 1e-5
_NEG_SLOPE = 0.2

_EALL = 330000
_CHUNK = 10320
_BATCH = 120
_NBF = 86
_NB31 = 84


def _gather_body(h_hbm, src_hbm, out_hbm, src_v, rows_v, sem):
    c = lax.axis_index("c")
    s = lax.axis_index("s")
    wid = s * 2 + c
    base = wid * _CHUNK
    is31 = wid == 31

    @pl.when(is31)
    def _():
        pltpu.sync_copy(src_hbm.at[pl.ds(base, 10080)],
                        src_v.at[pl.ds(0, 10080)])

    @pl.when(jnp.logical_not(is31))
    def _():
        pltpu.sync_copy(src_hbm.at[pl.ds(base, _CHUNK)], src_v)

    nb = jnp.where(is31, _NB31, _NBF)

    def batch(bi, carry):
        @pl.when(bi < nb)
        def _():
            off = bi * _BATCH
            idx = src_v.at[pl.ds(off, _BATCH)]
            pltpu.async_copy(h_hbm.at[idx], rows_v, sem).wait()
            pltpu.sync_copy(rows_v, out_hbm.at[pl.ds(base + off, _BATCH)])
        return carry

    lax.fori_loop(0, _NBF, batch, 0)


@functools.partial(
    pl.kernel,
    out_type=jax.ShapeDtypeStruct((_EALL, 128), jnp.float32),
    mesh=plsc.VectorSubcoreMesh(core_axis_name="c", subcore_axis_name="s"),
    compiler_params=pltpu.CompilerParams(needs_layout_passes=False),
    scratch_types=[
        pltpu.VMEM((_CHUNK,), jnp.int32),
        pltpu.VMEM((_BATCH, 128), jnp.float32),
        pltpu.SemaphoreType.DMA,
    ],
)
def _row_gather_kernel(h_hbm, src_hbm, out_hbm, src_v, rows_v, sem):
    _gather_body(h_hbm, src_hbm, out_hbm, src_v, rows_v, sem)


def _scale_body(a_ref, r_ref, o_ref):
    o_ref[...] = a_ref[...] * r_ref[...]


def _scale_rows(alpha, rows):
    blk = 5000
    grid = _EALL // blk
    return pl.pallas_call(
        _scale_body,
        grid=(grid,),
        in_specs=[
            pl.BlockSpec((blk, 1), lambda i: (i, 0)),
            pl.BlockSpec((blk, 128), lambda i: (i, 0)),
        ],
        out_specs=pl.BlockSpec((blk, 128), lambda i: (i, 0)),
        out_shape=jax.ShapeDtypeStruct((_EALL, 128), jnp.float32),
    )(alpha.reshape(-1, 1), rows)


def _mm_body(x_ref, w_ref, as_ref, ad_ref, h_ref, als_ref, ald_ref):
    h = jnp.dot(x_ref[...], w_ref[...], preferred_element_type=jnp.float32)
    h_ref[...] = h
    als_ref[...] = jnp.dot(h, as_ref[...], preferred_element_type=jnp.float32)
    ald_ref[...] = jnp.dot(h, ad_ref[...], preferred_element_type=jnp.float32)


def _pallas_pre(x, w, a_src, a_dst):
    n = x.shape[0]
    h, als, ald = pl.pallas_call(
        _mm_body,
        out_shape=[
            jax.ShapeDtypeStruct((n, w.shape[1]), jnp.float32),
            jax.ShapeDtypeStruct((n, 1), jnp.float32),
            jax.ShapeDtypeStruct((n, 1), jnp.float32),
        ],
    )(x, w, a_src.reshape(-1, 1), a_dst.reshape(-1, 1))
    return h, als.reshape(-1), ald.reshape(-1)


def _gat_conv(x, edge_index, W, a_src, a_dst, b):
    h, al_src_p, al_dst_p = _pallas_pre(x, W, a_src, a_dst)
    src = edge_index[0]
    dst = edge_index[1]
    loop = jnp.arange(_N, dtype=src.dtype)
    src = jnp.concatenate([src, loop])
    dst = jnp.concatenate([dst, loop])
    al_src = al_src_p
    al_dst = al_dst_p
    e = al_src[src] + al_dst[dst]
    e = _elemwise_1d(lambda t: jnp.where(t >= 0, t, _NEG_SLOPE * t), e)
    e_max = jax.ops.segment_max(e, dst, num_segments=_N)
    e = _elemwise_1d(lambda t, mx: jnp.exp(t - mx), e, e_max[dst])
    denom = jax.ops.segment_sum(e, dst, num_segments=_N)
    alpha = _elemwise_1d(lambda t, d: t / d, e, denom[dst])
    gathered = _row_gather_kernel(h, src)
    upd = _scale_rows(alpha, gathered)
    out = jax.ops.segment_sum(upd, dst, num_segments=_N)
    return out + b


def _batch_norm(x, g, b):
    m = jnp.mean(x, axis=0)
    v = jnp.var(x, axis=0)
    return g * (x - m) / jnp.sqrt(v + _EPS) + b


def _elemwise_1d(fn, *arrays):
    n = arrays[0].shape[0]
    rows = -(-n // 128)
    rows = -(-rows // 8) * 8
    np_ = rows * 128

    def body(*refs):
        out_ref = refs[-1]
        out_ref[...] = fn(*[r[...] for r in refs[:-1]])

    padded = [jnp.pad(a, (0, np_ - n)).reshape(rows, 128) for a in arrays]
    out = pl.pallas_call(
        body,
        out_shape=jax.ShapeDtypeStruct((rows, 128), jnp.float32),
    )(*padded)
    return out.reshape(-1)[:n]


def _final_body(h_ref, g_ref, b_ref, fw_ref, fb_ref, o_ref):
    x = h_ref[...]
    m = jnp.mean(x, axis=0)
    v = jnp.var(x, axis=0)
    hb = g_ref[...] * (x - m) / jnp.sqrt(v + _EPS) + b_ref[...]
    pooled = jnp.mean(hb, axis=0, keepdims=True)
    o_ref[...] = jnp.dot(pooled, fw_ref[...],
                         preferred_element_type=jnp.float32) + fb_ref[...]


def _pallas_final(h, g, b, fc_w, fc_b):
    out = pl.pallas_call(
        _final_body,
        out_shape=jax.ShapeDtypeStruct((1, 1), jnp.float32),
    )(h, g.reshape(1, -1), b.reshape(1, -1), fc_w, fc_b.reshape(1, 1))
    return out.reshape(-1)


def kernel(x, edge_index, W1, a_src1, a_dst1, b1, g1, be1, W2, a_src2, a_dst2, b2, g2, be2, fc_w, fc_b):
    h = _gat_conv(x, edge_index, W1, a_src1, a_dst1, b1)
    h = _batch_norm(h, g1, be1)
    h = jax.nn.relu(h)
    h = _gat_conv(h, edge_index, W2, a_src2, a_dst2, b2)
    m = jnp.mean(h, axis=0)
    v = jnp.var(h, axis=0)
    h = g2 * (h - m) / jnp.sqrt(v + _EPS) + be2
    pooled = jnp.mean(h, axis=0, keepdims=True)
    out = pooled @ fc_w + fc_b
    return out.reshape(-1)


# speedup vs baseline: 3.8968x; 3.3358x over previous
import functools

import jax
import jax.numpy as jnp
from jax import lax
from jax.experimental import pallas as pl
from jax.experimental.pallas import tpu as pltpu
from jax.experimental.pallas import tpu_sc as plsc

_N = 10000
_EPS = 1e-5
_NEG_SLOPE = 0.2

_EALL = 330000
_CHUNK = 10320
_BATCH = 120
_NBF = 86
_NB31 = 84


def _gather_body(h_hbm, src_hbm, out_hbm, src_v, rows_v, sem):
    c = lax.axis_index("c")
    s = lax.axis_index("s")
    wid = s * 2 + c
    base = wid * _CHUNK
    is31 = wid == 31

    @pl.when(is31)
    def _():
        pltpu.sync_copy(src_hbm.at[pl.ds(base, 10080)],
                        src_v.at[pl.ds(0, 10080)])

    @pl.when(jnp.logical_not(is31))
    def _():
        pltpu.sync_copy(src_hbm.at[pl.ds(base, _CHUNK)], src_v)

    nb = jnp.where(is31, _NB31, _NBF)

    def batch(bi, carry):
        @pl.when(bi < nb)
        def _():
            off = bi * _BATCH
            idx = src_v.at[pl.ds(off, _BATCH)]
            pltpu.async_copy(h_hbm.at[idx], rows_v, sem).wait()
            pltpu.sync_copy(rows_v, out_hbm.at[pl.ds(base + off, _BATCH)])
        return carry

    lax.fori_loop(0, _NBF, batch, 0)


@functools.partial(
    pl.kernel,
    out_type=jax.ShapeDtypeStruct((_EALL, 128), jnp.float32),
    mesh=plsc.VectorSubcoreMesh(core_axis_name="c", subcore_axis_name="s"),
    compiler_params=pltpu.CompilerParams(needs_layout_passes=False),
    scratch_types=[
        pltpu.VMEM((_CHUNK,), jnp.int32),
        pltpu.VMEM((_BATCH, 128), jnp.float32),
        pltpu.SemaphoreType.DMA,
    ],
)
def _row_gather_kernel(h_hbm, src_hbm, out_hbm, src_v, rows_v, sem):
    _gather_body(h_hbm, src_hbm, out_hbm, src_v, rows_v, sem)


def _make_edge_map(num_tabs, num_idx, num_lin, fn):
    _NVF = _CHUNK // 16
    _NV31 = 10080 // 16

    def body(*refs):
        tabs_hbm = refs[:num_tabs]
        idxs_hbm = refs[num_tabs:num_tabs + num_idx]
        lins_hbm = refs[num_tabs + num_idx:num_tabs + num_idx + num_lin]
        out_hbm = refs[num_tabs + num_idx + num_lin]
        sc = refs[num_tabs + num_idx + num_lin + 1:]
        tabs_v = sc[:num_tabs]
        idxs_v = sc[num_tabs:num_tabs + num_idx]
        lins_v = sc[num_tabs + num_idx:num_tabs + num_idx + num_lin]
        out_v = sc[num_tabs + num_idx + num_lin]

        c = lax.axis_index("c")
        s = lax.axis_index("s")
        wid = s * 2 + c
        base = wid * _CHUNK
        is31 = wid == 31

        for t_hbm, t_v in zip(tabs_hbm, tabs_v):
            pltpu.sync_copy(t_hbm, t_v)

        @pl.when(is31)
        def _():
            for a_hbm, a_v in zip(idxs_hbm + lins_hbm, idxs_v + lins_v):
                pltpu.sync_copy(a_hbm.at[pl.ds(base, 10080)],
                                a_v.at[pl.ds(0, 10080)])

        @pl.when(jnp.logical_not(is31))
        def _():
            for a_hbm, a_v in zip(idxs_hbm + lins_hbm, idxs_v + lins_v):
                pltpu.sync_copy(a_hbm.at[pl.ds(base, _CHUNK)], a_v)

        nv = jnp.where(is31, _NV31, _NVF)

        def step(k, carry):
            @pl.when(k < nv)
            def _():
                off = k * 16
                gs = [plsc.load_gather(t_v, [i_v[pl.ds(off, 16)]])
                      for t_v, i_v in zip(tabs_v, idxs_v)]
                ls = [l_v[pl.ds(off, 16)] for l_v in lins_v]
                out_v[pl.ds(off, 16)] = fn(*gs, *ls)
            return carry

        lax.fori_loop(0, _NVF, step, 0)

        @pl.when(is31)
        def _():
            pltpu.sync_copy(out_v.at[pl.ds(0, 10080)],
                            out_hbm.at[pl.ds(base, 10080)])

        @pl.when(jnp.logical_not(is31))
        def _():
            pltpu.sync_copy(out_v, out_hbm.at[pl.ds(base, _CHUNK)])

    scratch = ([pltpu.VMEM((_N,), jnp.float32)] * num_tabs
               + [pltpu.VMEM((_CHUNK,), jnp.int32)] * num_idx
               + [pltpu.VMEM((_CHUNK,), jnp.float32)] * num_lin
               + [pltpu.VMEM((_CHUNK,), jnp.float32)])
    return functools.partial(
        pl.kernel,
        out_type=jax.ShapeDtypeStruct((_EALL,), jnp.float32),
        mesh=plsc.VectorSubcoreMesh(core_axis_name="c", subcore_axis_name="s"),
        compiler_params=pltpu.CompilerParams(needs_layout_passes=False),
        scratch_types=scratch,
    )(body)


_edge_score_kernel = _make_edge_map(
    2, 2, 0, lambda a, bb: jnp.where(a + bb >= 0, a + bb, _NEG_SLOPE * (a + bb)))
_edge_sub_max_kernel = _make_edge_map(1, 1, 1, lambda mx, ee: ee - mx)
_edge_take_kernel = _make_edge_map(1, 1, 0, lambda dd: dd)


def _scale_body(a_ref, r_ref, o_ref):
    o_ref[...] = a_ref[...] * r_ref[...]


def _scale_rows(alpha, rows):
    blk = 5000
    grid = _EALL // blk
    return pl.pallas_call(
        _scale_body,
        grid=(grid,),
        in_specs=[
            pl.BlockSpec((blk, 1), lambda i: (i, 0)),
            pl.BlockSpec((blk, 128), lambda i: (i, 0)),
        ],
        out_specs=pl.BlockSpec((blk, 128), lambda i: (i, 0)),
        out_shape=jax.ShapeDtypeStruct((_EALL, 128), jnp.float32),
    )(alpha.reshape(-1, 1), rows)


def _mm_body(x_ref, w_ref, as_ref, ad_ref, h_ref, als_ref, ald_ref):
    h = jnp.dot(x_ref[...], w_ref[...], preferred_element_type=jnp.float32)
    h_ref[...] = h
    als_ref[...] = jnp.dot(h, as_ref[...], preferred_element_type=jnp.float32)
    ald_ref[...] = jnp.dot(h, ad_ref[...], preferred_element_type=jnp.float32)


def _pallas_pre(x, w, a_src, a_dst):
    n = x.shape[0]
    h, als, ald = pl.pallas_call(
        _mm_body,
        out_shape=[
            jax.ShapeDtypeStruct((n, w.shape[1]), jnp.float32),
            jax.ShapeDtypeStruct((n, 1), jnp.float32),
            jax.ShapeDtypeStruct((n, 1), jnp.float32),
        ],
    )(x, w, a_src.reshape(-1, 1), a_dst.reshape(-1, 1))
    return h, als.reshape(-1), ald.reshape(-1)


def _gat_conv(x, edge_index, W, a_src, a_dst, b):
    h, al_src_p, al_dst_p = _pallas_pre(x, W, a_src, a_dst)
    src = edge_index[0]
    dst = edge_index[1]
    loop = jnp.arange(_N, dtype=src.dtype)
    src = jnp.concatenate([src, loop])
    dst = jnp.concatenate([dst, loop])
    al_src = al_src_p
    al_dst = al_dst_p
    e = _edge_score_kernel(al_src, al_dst, src, dst)
    e_max = jax.ops.segment_max(e, dst, num_segments=_N)
    e = _elemwise_1d(jnp.exp, _edge_sub_max_kernel(e_max, dst, e))
    denom = jax.ops.segment_sum(e, dst, num_segments=_N)
    alpha = _elemwise_1d(lambda t, d: t / d, e, _edge_take_kernel(denom, dst))
    gathered = _row_gather_kernel(h, src)
    upd = _scale_rows(alpha, gathered)
    out = jax.ops.segment_sum(upd, dst, num_segments=_N)
    return out + b


def _batch_norm(x, g, b):
    m = jnp.mean(x, axis=0)
    v = jnp.var(x, axis=0)
    return g * (x - m) / jnp.sqrt(v + _EPS) + b


def _elemwise_1d(fn, *arrays):
    n = arrays[0].shape[0]
    rows = -(-n // 128)
    rows = -(-rows // 8) * 8
    np_ = rows * 128

    def body(*refs):
        out_ref = refs[-1]
        out_ref[...] = fn(*[r[...] for r in refs[:-1]])

    padded = [jnp.pad(a, (0, np_ - n)).reshape(rows, 128) for a in arrays]
    out = pl.pallas_call(
        body,
        out_shape=jax.ShapeDtypeStruct((rows, 128), jnp.float32),
    )(*padded)
    return out.reshape(-1)[:n]


def _final_body(h_ref, g_ref, b_ref, fw_ref, fb_ref, o_ref):
    x = h_ref[...]
    m = jnp.mean(x, axis=0)
    v = jnp.var(x, axis=0)
    hb = g_ref[...] * (x - m) / jnp.sqrt(v + _EPS) + b_ref[...]
    pooled = jnp.mean(hb, axis=0, keepdims=True)
    o_ref[...] = jnp.dot(pooled, fw_ref[...],
                         preferred_element_type=jnp.float32) + fb_ref[...]


def _pallas_final(h, g, b, fc_w, fc_b):
    out = pl.pallas_call(
        _final_body,
        out_shape=jax.ShapeDtypeStruct((1, 1), jnp.float32),
    )(h, g.reshape(1, -1), b.reshape(1, -1), fc_w, fc_b.reshape(1, 1))
    return out.reshape(-1)


def kernel(x, edge_index, W1, a_src1, a_dst1, b1, g1, be1, W2, a_src2, a_dst2, b2, g2, be2, fc_w, fc_b):
    h = _gat_conv(x, edge_index, W1, a_src1, a_dst1, b1)
    h = _batch_norm(h, g1, be1)
    h = jax.nn.relu(h)
    h = _gat_conv(h, edge_index, W2, a_src2, a_dst2, b2)
    m = jnp.mean(h, axis=0)
    v = jnp.var(h, axis=0)
    h = g2 * (h - m) / jnp.sqrt(v + _EPS) + be2
    pooled = jnp.mean(h, axis=0, keepdims=True)
    out = pooled @ fc_w + fc_b
    return out.reshape(-1)


# speedup vs baseline: 4.1533x; 1.0658x over previous
import functools

import jax
import jax.numpy as jnp
from jax import lax
from jax.experimental import pallas as pl
from jax.experimental.pallas import tpu as pltpu
from jax.experimental.pallas import tpu_sc as plsc

_N = 10000
_EPS = 1e-5
_NEG_SLOPE = 0.2

_EALL = 330000
_CHUNK = 10320
_BATCH = 120
_NBF = 86
_NB31 = 84


def _gather_body(h_hbm, src_hbm, alpha_hbm, out_hbm, src_v, alpha_v, rows_v, sem):
    c = lax.axis_index("c")
    s = lax.axis_index("s")
    wid = s * 2 + c
    base = wid * _CHUNK
    is31 = wid == 31

    @pl.when(is31)
    def _():
        pltpu.sync_copy(src_hbm.at[pl.ds(base, 10080)],
                        src_v.at[pl.ds(0, 10080)])
        pltpu.sync_copy(alpha_hbm.at[pl.ds(base, 10080)],
                        alpha_v.at[pl.ds(0, 10080)])

    @pl.when(jnp.logical_not(is31))
    def _():
        pltpu.sync_copy(src_hbm.at[pl.ds(base, _CHUNK)], src_v)
        pltpu.sync_copy(alpha_hbm.at[pl.ds(base, _CHUNK)], alpha_v)

    nb = jnp.where(is31, _NB31, _NBF)

    def batch(bi, carry):
        @pl.when(bi < nb)
        def _():
            off = bi * _BATCH
            idx = src_v.at[pl.ds(off, _BATCH)]
            pltpu.async_copy(h_hbm.at[idx], rows_v, sem).wait()

            def row(j, c2):
                ab = plsc.load_gather(
                    alpha_v, [jnp.full((16,), off + j, jnp.int32)])
                for vv in range(8):
                    sl = pl.ds(vv * 16, 16)
                    rows_v[j, sl] = rows_v[j, sl] * ab
                return c2

            lax.fori_loop(0, _BATCH, row, 0)
            pltpu.sync_copy(rows_v, out_hbm.at[pl.ds(base + off, _BATCH)])
        return carry

    lax.fori_loop(0, _NBF, batch, 0)


@functools.partial(
    pl.kernel,
    out_type=jax.ShapeDtypeStruct((_EALL, 128), jnp.float32),
    mesh=plsc.VectorSubcoreMesh(core_axis_name="c", subcore_axis_name="s"),
    compiler_params=pltpu.CompilerParams(needs_layout_passes=False),
    scratch_types=[
        pltpu.VMEM((_CHUNK,), jnp.int32),
        pltpu.VMEM((_CHUNK,), jnp.float32),
        pltpu.VMEM((_BATCH, 128), jnp.float32),
        pltpu.SemaphoreType.DMA,
    ],
)
def _row_gather_kernel(h_hbm, src_hbm, alpha_hbm, out_hbm, src_v, alpha_v,
                       rows_v, sem):
    _gather_body(h_hbm, src_hbm, alpha_hbm, out_hbm, src_v, alpha_v, rows_v,
                 sem)


def _make_edge_map(num_tabs, num_idx, num_lin, fn):
    _NVF = _CHUNK // 16
    _NV31 = 10080 // 16

    def body(*refs):
        tabs_hbm = refs[:num_tabs]
        idxs_hbm = refs[num_tabs:num_tabs + num_idx]
        lins_hbm = refs[num_tabs + num_idx:num_tabs + num_idx + num_lin]
        out_hbm = refs[num_tabs + num_idx + num_lin]
        sc = refs[num_tabs + num_idx + num_lin + 1:]
        tabs_v = sc[:num_tabs]
        idxs_v = sc[num_tabs:num_tabs + num_idx]
        lins_v = sc[num_tabs + num_idx:num_tabs + num_idx + num_lin]
        out_v = sc[num_tabs + num_idx + num_lin]

        c = lax.axis_index("c")
        s = lax.axis_index("s")
        wid = s * 2 + c
        base = wid * _CHUNK
        is31 = wid == 31

        for t_hbm, t_v in zip(tabs_hbm, tabs_v):
            pltpu.sync_copy(t_hbm, t_v)

        @pl.when(is31)
        def _():
            for a_hbm, a_v in zip(idxs_hbm + lins_hbm, idxs_v + lins_v):
                pltpu.sync_copy(a_hbm.at[pl.ds(base, 10080)],
                                a_v.at[pl.ds(0, 10080)])

        @pl.when(jnp.logical_not(is31))
        def _():
            for a_hbm, a_v in zip(idxs_hbm + lins_hbm, idxs_v + lins_v):
                pltpu.sync_copy(a_hbm.at[pl.ds(base, _CHUNK)], a_v)

        nv = jnp.where(is31, _NV31, _NVF)

        def step(k, carry):
            @pl.when(k < nv)
            def _():
                off = k * 16
                gs = [plsc.load_gather(t_v, [i_v[pl.ds(off, 16)]])
                      for t_v, i_v in zip(tabs_v, idxs_v)]
                ls = [l_v[pl.ds(off, 16)] for l_v in lins_v]
                out_v[pl.ds(off, 16)] = fn(*gs, *ls)
            return carry

        lax.fori_loop(0, _NVF, step, 0)

        @pl.when(is31)
        def _():
            pltpu.sync_copy(out_v.at[pl.ds(0, 10080)],
                            out_hbm.at[pl.ds(base, 10080)])

        @pl.when(jnp.logical_not(is31))
        def _():
            pltpu.sync_copy(out_v, out_hbm.at[pl.ds(base, _CHUNK)])

    scratch = ([pltpu.VMEM((_N,), jnp.float32)] * num_tabs
               + [pltpu.VMEM((_CHUNK,), jnp.int32)] * num_idx
               + [pltpu.VMEM((_CHUNK,), jnp.float32)] * num_lin
               + [pltpu.VMEM((_CHUNK,), jnp.float32)])
    return functools.partial(
        pl.kernel,
        out_type=jax.ShapeDtypeStruct((_EALL,), jnp.float32),
        mesh=plsc.VectorSubcoreMesh(core_axis_name="c", subcore_axis_name="s"),
        compiler_params=pltpu.CompilerParams(needs_layout_passes=False),
        scratch_types=scratch,
    )(body)


_edge_score_kernel = _make_edge_map(
    2, 2, 0, lambda a, bb: jnp.where(a + bb >= 0, a + bb, _NEG_SLOPE * (a + bb)))
_edge_sub_max_kernel = _make_edge_map(1, 1, 1, lambda mx, ee: ee - mx)
_edge_take_kernel = _make_edge_map(1, 1, 0, lambda dd: dd)


def _scale_body(a_ref, r_ref, o_ref):
    o_ref[...] = a_ref[...] * r_ref[...]


def _scale_rows(alpha, rows):
    blk = 5000
    grid = _EALL // blk
    return pl.pallas_call(
        _scale_body,
        grid=(grid,),
        in_specs=[
            pl.BlockSpec((blk, 1), lambda i: (i, 0)),
            pl.BlockSpec((blk, 128), lambda i: (i, 0)),
        ],
        out_specs=pl.BlockSpec((blk, 128), lambda i: (i, 0)),
        out_shape=jax.ShapeDtypeStruct((_EALL, 128), jnp.float32),
    )(alpha.reshape(-1, 1), rows)


def _mm_body(x_ref, w_ref, as_ref, ad_ref, h_ref, als_ref, ald_ref):
    h = jnp.dot(x_ref[...], w_ref[...], preferred_element_type=jnp.float32)
    h_ref[...] = h
    als_ref[...] = jnp.dot(h, as_ref[...], preferred_element_type=jnp.float32)
    ald_ref[...] = jnp.dot(h, ad_ref[...], preferred_element_type=jnp.float32)


def _pallas_pre(x, w, a_src, a_dst):
    n = x.shape[0]
    h, als, ald = pl.pallas_call(
        _mm_body,
        out_shape=[
            jax.ShapeDtypeStruct((n, w.shape[1]), jnp.float32),
            jax.ShapeDtypeStruct((n, 1), jnp.float32),
            jax.ShapeDtypeStruct((n, 1), jnp.float32),
        ],
    )(x, w, a_src.reshape(-1, 1), a_dst.reshape(-1, 1))
    return h, als.reshape(-1), ald.reshape(-1)


def _gat_conv(x, edge_index, W, a_src, a_dst, b):
    h, al_src_p, al_dst_p = _pallas_pre(x, W, a_src, a_dst)
    src = edge_index[0]
    dst = edge_index[1]
    loop = jnp.arange(_N, dtype=src.dtype)
    src = jnp.concatenate([src, loop])
    dst = jnp.concatenate([dst, loop])
    al_src = al_src_p
    al_dst = al_dst_p
    e = _edge_score_kernel(al_src, al_dst, src, dst)
    e_max = jax.ops.segment_max(e, dst, num_segments=_N)
    e = _elemwise_1d(jnp.exp, _edge_sub_max_kernel(e_max, dst, e))
    denom = jax.ops.segment_sum(e, dst, num_segments=_N)
    alpha = _elemwise_1d(lambda t, d: t / d, e, _edge_take_kernel(denom, dst))
    upd = _row_gather_kernel(h, src, alpha)
    out = jax.ops.segment_sum(upd, dst, num_segments=_N)
    return out + b


def _batch_norm(x, g, b):
    m = jnp.mean(x, axis=0)
    v = jnp.var(x, axis=0)
    return g * (x - m) / jnp.sqrt(v + _EPS) + b


def _elemwise_1d(fn, *arrays):
    n = arrays[0].shape[0]
    rows = -(-n // 128)
    rows = -(-rows // 8) * 8
    np_ = rows * 128

    def body(*refs):
        out_ref = refs[-1]
        out_ref[...] = fn(*[r[...] for r in refs[:-1]])

    padded = [jnp.pad(a, (0, np_ - n)).reshape(rows, 128) for a in arrays]
    out = pl.pallas_call(
        body,
        out_shape=jax.ShapeDtypeStruct((rows, 128), jnp.float32),
    )(*padded)
    return out.reshape(-1)[:n]


def _final_body(h_ref, g_ref, b_ref, fw_ref, fb_ref, o_ref):
    x = h_ref[...]
    m = jnp.mean(x, axis=0)
    v = jnp.var(x, axis=0)
    hb = g_ref[...] * (x - m) / jnp.sqrt(v + _EPS) + b_ref[...]
    pooled = jnp.mean(hb, axis=0, keepdims=True)
    o_ref[...] = jnp.dot(pooled, fw_ref[...],
                         preferred_element_type=jnp.float32) + fb_ref[...]


def _pallas_final(h, g, b, fc_w, fc_b):
    out = pl.pallas_call(
        _final_body,
        out_shape=jax.ShapeDtypeStruct((1, 1), jnp.float32),
    )(h, g.reshape(1, -1), b.reshape(1, -1), fc_w, fc_b.reshape(1, 1))
    return out.reshape(-1)


def kernel(x, edge_index, W1, a_src1, a_dst1, b1, g1, be1, W2, a_src2, a_dst2, b2, g2, be2, fc_w, fc_b):
    h = _gat_conv(x, edge_index, W1, a_src1, a_dst1, b1)
    h = _batch_norm(h, g1, be1)
    h = jax.nn.relu(h)
    h = _gat_conv(h, edge_index, W2, a_src2, a_dst2, b2)
    m = jnp.mean(h, axis=0)
    v = jnp.var(h, axis=0)
    h = g2 * (h - m) / jnp.sqrt(v + _EPS) + be2
    pooled = jnp.mean(h, axis=0, keepdims=True)
    out = pooled @ fc_w + fc_b
    return out.reshape(-1)


# speedup vs baseline: 5.1985x; 1.2516x over previous
import functools

import jax
import jax.numpy as jnp
from jax import lax
from jax.experimental import pallas as pl
from jax.experimental.pallas import tpu as pltpu
from jax.experimental.pallas import tpu_sc as plsc

_N = 10000
_EPS = 1e-5
_NEG_SLOPE = 0.2

_EALL = 330000
_CHUNK = 10320
_BATCH = 120
_NBF = 86
_NB31 = 84


def _gather_body(h_hbm, src_hbm, alpha_hbm, out_hbm, src_v, alpha_v, rows_v, sem):
    c = lax.axis_index("c")
    s = lax.axis_index("s")
    wid = s * 2 + c
    base = wid * _CHUNK
    is31 = wid == 31

    @pl.when(is31)
    def _():
        pltpu.sync_copy(src_hbm.at[pl.ds(base, 10080)],
                        src_v.at[pl.ds(0, 10080)])
        pltpu.sync_copy(alpha_hbm.at[pl.ds(base, 10080)],
                        alpha_v.at[pl.ds(0, 10080)])

    @pl.when(jnp.logical_not(is31))
    def _():
        pltpu.sync_copy(src_hbm.at[pl.ds(base, _CHUNK)], src_v)
        pltpu.sync_copy(alpha_hbm.at[pl.ds(base, _CHUNK)], alpha_v)

    nb = jnp.where(is31, _NB31, _NBF)

    def batch(bi, carry):
        @pl.when(bi < nb)
        def _():
            off = bi * _BATCH
            idx = src_v.at[pl.ds(off, _BATCH)]
            pltpu.async_copy(h_hbm.at[idx], rows_v, sem).wait()

            def row(j, c2):
                ab = plsc.load_gather(
                    alpha_v, [jnp.full((16,), off + j, jnp.int32)])
                for vv in range(8):
                    sl = pl.ds(vv * 16, 16)
                    rows_v[j, sl] = rows_v[j, sl] * ab
                return c2

            lax.fori_loop(0, _BATCH, row, 0)
            pltpu.sync_copy(rows_v, out_hbm.at[pl.ds(base + off, _BATCH)])
        return carry

    lax.fori_loop(0, _NBF, batch, 0)


@functools.partial(
    pl.kernel,
    out_type=jax.ShapeDtypeStruct((_EALL, 128), jnp.float32),
    mesh=plsc.VectorSubcoreMesh(core_axis_name="c", subcore_axis_name="s"),
    compiler_params=pltpu.CompilerParams(needs_layout_passes=False),
    scratch_types=[
        pltpu.VMEM((_CHUNK,), jnp.int32),
        pltpu.VMEM((_CHUNK,), jnp.float32),
        pltpu.VMEM((_BATCH, 128), jnp.float32),
        pltpu.SemaphoreType.DMA,
    ],
)
def _row_gather_kernel(h_hbm, src_hbm, alpha_hbm, out_hbm, src_v, alpha_v,
                       rows_v, sem):
    _gather_body(h_hbm, src_hbm, alpha_hbm, out_hbm, src_v, alpha_v, rows_v,
                 sem)


def _make_edge_map(num_tabs, num_idx, num_lin, fn):
    _NVF = _CHUNK // 16
    _NV31 = 10080 // 16

    def body(*refs):
        tabs_hbm = refs[:num_tabs]
        idxs_hbm = refs[num_tabs:num_tabs + num_idx]
        lins_hbm = refs[num_tabs + num_idx:num_tabs + num_idx + num_lin]
        out_hbm = refs[num_tabs + num_idx + num_lin]
        sc = refs[num_tabs + num_idx + num_lin + 1:]
        tabs_v = sc[:num_tabs]
        idxs_v = sc[num_tabs:num_tabs + num_idx]
        lins_v = sc[num_tabs + num_idx:num_tabs + num_idx + num_lin]
        out_v = sc[num_tabs + num_idx + num_lin]

        c = lax.axis_index("c")
        s = lax.axis_index("s")
        wid = s * 2 + c
        base = wid * _CHUNK
        is31 = wid == 31

        for t_hbm, t_v in zip(tabs_hbm, tabs_v):
            pltpu.sync_copy(t_hbm, t_v)

        @pl.when(is31)
        def _():
            for a_hbm, a_v in zip(idxs_hbm + lins_hbm, idxs_v + lins_v):
                pltpu.sync_copy(a_hbm.at[pl.ds(base, 10080)],
                                a_v.at[pl.ds(0, 10080)])

        @pl.when(jnp.logical_not(is31))
        def _():
            for a_hbm, a_v in zip(idxs_hbm + lins_hbm, idxs_v + lins_v):
                pltpu.sync_copy(a_hbm.at[pl.ds(base, _CHUNK)], a_v)

        nv = jnp.where(is31, _NV31, _NVF)

        def step(k, carry):
            @pl.when(k < nv)
            def _():
                off = k * 16
                gs = [plsc.load_gather(t_v, [i_v[pl.ds(off, 16)]])
                      for t_v, i_v in zip(tabs_v, idxs_v)]
                ls = [l_v[pl.ds(off, 16)] for l_v in lins_v]
                out_v[pl.ds(off, 16)] = fn(*gs, *ls)
            return carry

        lax.fori_loop(0, _NVF, step, 0)

        @pl.when(is31)
        def _():
            pltpu.sync_copy(out_v.at[pl.ds(0, 10080)],
                            out_hbm.at[pl.ds(base, 10080)])

        @pl.when(jnp.logical_not(is31))
        def _():
            pltpu.sync_copy(out_v, out_hbm.at[pl.ds(base, _CHUNK)])

    scratch = ([pltpu.VMEM((_N,), jnp.float32)] * num_tabs
               + [pltpu.VMEM((_CHUNK,), jnp.int32)] * num_idx
               + [pltpu.VMEM((_CHUNK,), jnp.float32)] * num_lin
               + [pltpu.VMEM((_CHUNK,), jnp.float32)])
    return functools.partial(
        pl.kernel,
        out_type=jax.ShapeDtypeStruct((_EALL,), jnp.float32),
        mesh=plsc.VectorSubcoreMesh(core_axis_name="c", subcore_axis_name="s"),
        compiler_params=pltpu.CompilerParams(needs_layout_passes=False),
        scratch_types=scratch,
    )(body)


_edge_score_kernel = _make_edge_map(
    2, 2, 0, lambda a, bb: jnp.where(a + bb >= 0, a + bb, _NEG_SLOPE * (a + bb)))
_edge_sub_max_kernel = _make_edge_map(1, 1, 1, lambda mx, ee: ee - mx)
_edge_take_kernel = _make_edge_map(1, 1, 0, lambda dd: dd)


_QCAP = 16512


def _iota16():
    return lax.iota(jnp.int32, 16)


def _rowsum_body(upd_hbm, dst_hbm, out_hbm, dstc_v, idq_v, dlq_v, rows_v,
                 acc_v, sem):
    c = lax.axis_index("c")
    s = lax.axis_index("s")
    wid = s * 2 + c
    npt = jnp.where(wid < 2, 320, 312)
    lo = 320 * jnp.minimum(wid, 2) + 312 * jnp.maximum(wid - 2, 0)
    lo16 = jnp.full((16,), lo, jnp.int32)
    hi16 = lo16 + npt

    def zrow(r, cz):
        for vv in range(8):
            acc_v[r, pl.ds(vv * 16, 16)] = jnp.zeros((16,), jnp.float32)
        return cz
    lax.fori_loop(0, 320, zrow, 0)

    def zq(r, cz):
        idq_v[pl.ds(r * 16, 16)] = jnp.zeros((16,), jnp.int32)
        return cz
    lax.fori_loop(0, _QCAP // 16, zq, 0)

    def scan_chunk(ci, qn):
        base_e = ci * _CHUNK
        is31 = ci == 31

        @pl.when(is31)
        def _():
            pltpu.sync_copy(dst_hbm.at[pl.ds(base_e, 10080)],
                            dstc_v.at[pl.ds(0, 10080)])

        @pl.when(jnp.logical_not(is31))
        def _():
            pltpu.sync_copy(dst_hbm.at[pl.ds(base_e, _CHUNK)], dstc_v)

        nv = jnp.where(is31, 10080 // 16, _CHUNK // 16)

        def step(k, qn2):
            d16 = dstc_v[pl.ds(k * 16, 16)]
            valid = jnp.full((16,), k < nv, jnp.bool_)
            m = (d16 >= lo16) & (d16 < hi16) & valid
            ids = jnp.full((16,), base_e + k * 16, jnp.int32) + _iota16()
            cnt = jnp.sum(jnp.where(m, 1, 0))
            plsc.store_compressed(idq_v.at[pl.ds(qn2, 16)], ids, mask=m)
            plsc.store_compressed(dlq_v.at[pl.ds(qn2, 16)], d16 - lo16, mask=m)
            return qn2 + cnt

        return lax.fori_loop(0, _CHUNK // 16, step, qn)

    qn = lax.fori_loop(0, 32, scan_chunk, jnp.int32(0))

    def batch(b, cz):
        @pl.when(b * _BATCH < qn)
        def _():
            idx = idq_v.at[pl.ds(b * _BATCH, _BATCH)]
            pltpu.async_copy(upd_hbm.at[idx], rows_v, sem).wait()

            def row(j, cr):
                qi = b * _BATCH + j

                @pl.when(qi < qn)
                def _():
                    dlv = plsc.load_gather(
                        dlq_v, [jnp.full((16,), qi, jnp.int32)])
                    for vv in range(8):
                        col = _iota16() + (vv * 16)
                        old = plsc.load_gather(acc_v, [dlv, col])
                        plsc.store_scatter(
                            acc_v, [dlv, col],
                            old + rows_v[j, pl.ds(vv * 16, 16)])
                return cr

            lax.fori_loop(0, _BATCH, row, 0)
        return cz

    lax.fori_loop(0, _QCAP // _BATCH, batch, 0)

    @pl.when(wid < 2)
    def _():
        pltpu.sync_copy(acc_v.at[pl.ds(0, 320)], out_hbm.at[pl.ds(lo, 320)])

    @pl.when(wid >= 2)
    def _():
        pltpu.sync_copy(acc_v.at[pl.ds(0, 312)], out_hbm.at[pl.ds(lo, 312)])


@functools.partial(
    pl.kernel,
    out_type=jax.ShapeDtypeStruct((_N, 128), jnp.float32),
    mesh=plsc.VectorSubcoreMesh(core_axis_name="c", subcore_axis_name="s"),
    compiler_params=pltpu.CompilerParams(needs_layout_passes=False),
    scratch_types=[
        pltpu.VMEM((_CHUNK,), jnp.int32),
        pltpu.VMEM((_QCAP,), jnp.int32),
        pltpu.VMEM((_QCAP,), jnp.int32),
        pltpu.VMEM((_BATCH, 128), jnp.float32),
        pltpu.VMEM((320, 128), jnp.float32),
        pltpu.SemaphoreType.DMA,
    ],
)
def _seg_rowsum_kernel(upd_hbm, dst_hbm, out_hbm, dstc_v, idq_v, dlq_v,
                       rows_v, acc_v, sem):
    _rowsum_body(upd_hbm, dst_hbm, out_hbm, dstc_v, idq_v, dlq_v, rows_v,
                 acc_v, sem)


def _scale_body(a_ref, r_ref, o_ref):
    o_ref[...] = a_ref[...] * r_ref[...]


def _scale_rows(alpha, rows):
    blk = 5000
    grid = _EALL // blk
    return pl.pallas_call(
        _scale_body,
        grid=(grid,),
        in_specs=[
            pl.BlockSpec((blk, 1), lambda i: (i, 0)),
            pl.BlockSpec((blk, 128), lambda i: (i, 0)),
        ],
        out_specs=pl.BlockSpec((blk, 128), lambda i: (i, 0)),
        out_shape=jax.ShapeDtypeStruct((_EALL, 128), jnp.float32),
    )(alpha.reshape(-1, 1), rows)


def _mm_body(x_ref, w_ref, as_ref, ad_ref, h_ref, als_ref, ald_ref):
    h = jnp.dot(x_ref[...], w_ref[...], preferred_element_type=jnp.float32)
    h_ref[...] = h
    als_ref[...] = jnp.dot(h, as_ref[...], preferred_element_type=jnp.float32)
    ald_ref[...] = jnp.dot(h, ad_ref[...], preferred_element_type=jnp.float32)


def _pallas_pre(x, w, a_src, a_dst):
    n = x.shape[0]
    h, als, ald = pl.pallas_call(
        _mm_body,
        out_shape=[
            jax.ShapeDtypeStruct((n, w.shape[1]), jnp.float32),
            jax.ShapeDtypeStruct((n, 1), jnp.float32),
            jax.ShapeDtypeStruct((n, 1), jnp.float32),
        ],
    )(x, w, a_src.reshape(-1, 1), a_dst.reshape(-1, 1))
    return h, als.reshape(-1), ald.reshape(-1)


def _gat_conv(x, edge_index, W, a_src, a_dst, b):
    h, al_src_p, al_dst_p = _pallas_pre(x, W, a_src, a_dst)
    src = edge_index[0]
    dst = edge_index[1]
    loop = jnp.arange(_N, dtype=src.dtype)
    src = jnp.concatenate([src, loop])
    dst = jnp.concatenate([dst, loop])
    al_src = al_src_p
    al_dst = al_dst_p
    e = _edge_score_kernel(al_src, al_dst, src, dst)
    e_max = jax.ops.segment_max(e, dst, num_segments=_N)
    e = _elemwise_1d(jnp.exp, _edge_sub_max_kernel(e_max, dst, e))
    denom = jax.ops.segment_sum(e, dst, num_segments=_N)
    alpha = _elemwise_1d(lambda t, d: t / d, e, _edge_take_kernel(denom, dst))
    upd = _row_gather_kernel(h, src, alpha)
    out = _seg_rowsum_kernel(upd, dst)
    return out + b


def _batch_norm(x, g, b):
    m = jnp.mean(x, axis=0)
    v = jnp.var(x, axis=0)
    return g * (x - m) / jnp.sqrt(v + _EPS) + b


def _elemwise_1d(fn, *arrays):
    n = arrays[0].shape[0]
    rows = -(-n // 128)
    rows = -(-rows // 8) * 8
    np_ = rows * 128

    def body(*refs):
        out_ref = refs[-1]
        out_ref[...] = fn(*[r[...] for r in refs[:-1]])

    padded = [jnp.pad(a, (0, np_ - n)).reshape(rows, 128) for a in arrays]
    out = pl.pallas_call(
        body,
        out_shape=jax.ShapeDtypeStruct((rows, 128), jnp.float32),
    )(*padded)
    return out.reshape(-1)[:n]


def _final_body(h_ref, g_ref, b_ref, fw_ref, fb_ref, o_ref):
    x = h_ref[...]
    m = jnp.mean(x, axis=0)
    v = jnp.var(x, axis=0)
    hb = g_ref[...] * (x - m) / jnp.sqrt(v + _EPS) + b_ref[...]
    pooled = jnp.mean(hb, axis=0, keepdims=True)
    o_ref[...] = jnp.dot(pooled, fw_ref[...],
                         preferred_element_type=jnp.float32) + fb_ref[...]


def _pallas_final(h, g, b, fc_w, fc_b):
    out = pl.pallas_call(
        _final_body,
        out_shape=jax.ShapeDtypeStruct((1, 1), jnp.float32),
    )(h, g.reshape(1, -1), b.reshape(1, -1), fc_w, fc_b.reshape(1, 1))
    return out.reshape(-1)


def kernel(x, edge_index, W1, a_src1, a_dst1, b1, g1, be1, W2, a_src2, a_dst2, b2, g2, be2, fc_w, fc_b):
    h = _gat_conv(x, edge_index, W1, a_src1, a_dst1, b1)
    h = _batch_norm(h, g1, be1)
    h = jax.nn.relu(h)
    h = _gat_conv(h, edge_index, W2, a_src2, a_dst2, b2)
    m = jnp.mean(h, axis=0)
    v = jnp.var(h, axis=0)
    h = g2 * (h - m) / jnp.sqrt(v + _EPS) + be2
    pooled = jnp.mean(h, axis=0, keepdims=True)
    out = pooled @ fc_w + fc_b
    return out.reshape(-1)


# speedup vs baseline: 5.2203x; 1.0042x over previous
import functools

import jax
import jax.numpy as jnp
from jax import lax
from jax.experimental import pallas as pl
from jax.experimental.pallas import tpu as pltpu
from jax.experimental.pallas import tpu_sc as plsc

_N = 10000
_EPS = 1e-5
_NEG_SLOPE = 0.2

_EALL = 330000
_CHUNK = 10320
_BATCH = 120
_NBF = 86
_NB31 = 84


def _gather_body(h_hbm, src_hbm, alpha_hbm, out_hbm, src_v, alpha_v, rows_v, sem):
    c = lax.axis_index("c")
    s = lax.axis_index("s")
    wid = s * 2 + c
    base = wid * _CHUNK
    is31 = wid == 31

    @pl.when(is31)
    def _():
        pltpu.sync_copy(src_hbm.at[pl.ds(base, 10080)],
                        src_v.at[pl.ds(0, 10080)])
        pltpu.sync_copy(alpha_hbm.at[pl.ds(base, 10080)],
                        alpha_v.at[pl.ds(0, 10080)])

    @pl.when(jnp.logical_not(is31))
    def _():
        pltpu.sync_copy(src_hbm.at[pl.ds(base, _CHUNK)], src_v)
        pltpu.sync_copy(alpha_hbm.at[pl.ds(base, _CHUNK)], alpha_v)

    nb = jnp.where(is31, _NB31, _NBF)

    def batch(bi, carry):
        @pl.when(bi < nb)
        def _():
            off = bi * _BATCH
            idx = src_v.at[pl.ds(off, _BATCH)]
            pltpu.async_copy(h_hbm.at[idx], rows_v, sem).wait()

            def row(j, c2):
                ab = plsc.load_gather(
                    alpha_v, [jnp.full((16,), off + j, jnp.int32)])
                for vv in range(8):
                    sl = pl.ds(vv * 16, 16)
                    rows_v[j, sl] = rows_v[j, sl] * ab
                return c2

            lax.fori_loop(0, _BATCH, row, 0)
            pltpu.sync_copy(rows_v, out_hbm.at[pl.ds(base + off, _BATCH)])
        return carry

    lax.fori_loop(0, _NBF, batch, 0)


@functools.partial(
    pl.kernel,
    out_type=jax.ShapeDtypeStruct((_EALL, 128), jnp.float32),
    mesh=plsc.VectorSubcoreMesh(core_axis_name="c", subcore_axis_name="s"),
    compiler_params=pltpu.CompilerParams(needs_layout_passes=False),
    scratch_types=[
        pltpu.VMEM((_CHUNK,), jnp.int32),
        pltpu.VMEM((_CHUNK,), jnp.float32),
        pltpu.VMEM((_BATCH, 128), jnp.float32),
        pltpu.SemaphoreType.DMA,
    ],
)
def _row_gather_kernel(h_hbm, src_hbm, alpha_hbm, out_hbm, src_v, alpha_v,
                       rows_v, sem):
    _gather_body(h_hbm, src_hbm, alpha_hbm, out_hbm, src_v, alpha_v, rows_v,
                 sem)


def _make_edge_map(num_tabs, num_idx, num_lin, fn):
    _NVF = _CHUNK // 16
    _NV31 = 10080 // 16

    def body(*refs):
        tabs_hbm = refs[:num_tabs]
        idxs_hbm = refs[num_tabs:num_tabs + num_idx]
        lins_hbm = refs[num_tabs + num_idx:num_tabs + num_idx + num_lin]
        out_hbm = refs[num_tabs + num_idx + num_lin]
        sc = refs[num_tabs + num_idx + num_lin + 1:]
        tabs_v = sc[:num_tabs]
        idxs_v = sc[num_tabs:num_tabs + num_idx]
        lins_v = sc[num_tabs + num_idx:num_tabs + num_idx + num_lin]
        out_v = sc[num_tabs + num_idx + num_lin]

        c = lax.axis_index("c")
        s = lax.axis_index("s")
        wid = s * 2 + c
        base = wid * _CHUNK
        is31 = wid == 31

        for t_hbm, t_v in zip(tabs_hbm, tabs_v):
            pltpu.sync_copy(t_hbm, t_v)

        @pl.when(is31)
        def _():
            for a_hbm, a_v in zip(idxs_hbm + lins_hbm, idxs_v + lins_v):
                pltpu.sync_copy(a_hbm.at[pl.ds(base, 10080)],
                                a_v.at[pl.ds(0, 10080)])

        @pl.when(jnp.logical_not(is31))
        def _():
            for a_hbm, a_v in zip(idxs_hbm + lins_hbm, idxs_v + lins_v):
                pltpu.sync_copy(a_hbm.at[pl.ds(base, _CHUNK)], a_v)

        nv = jnp.where(is31, _NV31, _NVF)

        def step(k, carry):
            @pl.when(k < nv)
            def _():
                off = k * 16
                gs = [plsc.load_gather(t_v, [i_v[pl.ds(off, 16)]])
                      for t_v, i_v in zip(tabs_v, idxs_v)]
                ls = [l_v[pl.ds(off, 16)] for l_v in lins_v]
                out_v[pl.ds(off, 16)] = fn(*gs, *ls)
            return carry

        lax.fori_loop(0, _NVF, step, 0)

        @pl.when(is31)
        def _():
            pltpu.sync_copy(out_v.at[pl.ds(0, 10080)],
                            out_hbm.at[pl.ds(base, 10080)])

        @pl.when(jnp.logical_not(is31))
        def _():
            pltpu.sync_copy(out_v, out_hbm.at[pl.ds(base, _CHUNK)])

    scratch = ([pltpu.VMEM((_N,), jnp.float32)] * num_tabs
               + [pltpu.VMEM((_CHUNK,), jnp.int32)] * num_idx
               + [pltpu.VMEM((_CHUNK,), jnp.float32)] * num_lin
               + [pltpu.VMEM((_CHUNK,), jnp.float32)])
    return functools.partial(
        pl.kernel,
        out_type=jax.ShapeDtypeStruct((_EALL,), jnp.float32),
        mesh=plsc.VectorSubcoreMesh(core_axis_name="c", subcore_axis_name="s"),
        compiler_params=pltpu.CompilerParams(needs_layout_passes=False),
        scratch_types=scratch,
    )(body)


_edge_score_kernel = _make_edge_map(
    2, 2, 0, lambda a, bb: jnp.where(a + bb >= 0, a + bb, _NEG_SLOPE * (a + bb)))
_edge_sub_max_kernel = _make_edge_map(1, 1, 1, lambda mx, ee: ee - mx)
_edge_take_kernel = _make_edge_map(1, 1, 0, lambda dd: dd)


_QCAP = 16512


def _iota16():
    return lax.iota(jnp.int32, 16)


def _rowsum_body(upd_hbm, dst_hbm, out_hbm, dstc_v, idq_v, dlq_v, rows_v,
                 acc_v, sem):
    c = lax.axis_index("c")
    s = lax.axis_index("s")
    wid = s * 2 + c
    npt = jnp.where(wid < 2, 320, 312)
    lo = 320 * jnp.minimum(wid, 2) + 312 * jnp.maximum(wid - 2, 0)
    lo16 = jnp.full((16,), lo, jnp.int32)
    hi16 = lo16 + npt

    def zrow(r, cz):
        for vv in range(8):
            acc_v[r, pl.ds(vv * 16, 16)] = jnp.zeros((16,), jnp.float32)
        return cz
    lax.fori_loop(0, 320, zrow, 0)

    def zq(r, cz):
        idq_v[pl.ds(r * 16, 16)] = jnp.zeros((16,), jnp.int32)
        return cz
    lax.fori_loop(0, _QCAP // 16, zq, 0)

    def scan_chunk(ci, qn):
        base_e = ci * _CHUNK
        is31 = ci == 31

        @pl.when(is31)
        def _():
            pltpu.sync_copy(dst_hbm.at[pl.ds(base_e, 10080)],
                            dstc_v.at[pl.ds(0, 10080)])

        @pl.when(jnp.logical_not(is31))
        def _():
            pltpu.sync_copy(dst_hbm.at[pl.ds(base_e, _CHUNK)], dstc_v)

        nv = jnp.where(is31, 10080 // 16, _CHUNK // 16)

        def step(k, qn2):
            d16 = dstc_v[pl.ds(k * 16, 16)]
            valid = jnp.full((16,), k < nv, jnp.bool_)
            m = (d16 >= lo16) & (d16 < hi16) & valid
            ids = jnp.full((16,), base_e + k * 16, jnp.int32) + _iota16()
            cnt = jnp.sum(jnp.where(m, 1, 0))
            plsc.store_compressed(idq_v.at[pl.ds(qn2, 16)], ids, mask=m)
            plsc.store_compressed(dlq_v.at[pl.ds(qn2, 16)], d16 - lo16, mask=m)
            return qn2 + cnt

        return lax.fori_loop(0, _CHUNK // 16, step, qn)

    qn = lax.fori_loop(0, 32, scan_chunk, jnp.int32(0))

    def batch(b, cz):
        @pl.when(b * _BATCH < qn)
        def _():
            idx = idq_v.at[pl.ds(b * _BATCH, _BATCH)]
            pltpu.async_copy(upd_hbm.at[idx], rows_v, sem).wait()

            def row(j, cr):
                qi = b * _BATCH + j

                @pl.when(qi < qn)
                def _():
                    dlv = plsc.load_gather(
                        dlq_v, [jnp.full((16,), qi, jnp.int32)])
                    for vv in range(8):
                        col = _iota16() + (vv * 16)
                        old = plsc.load_gather(acc_v, [dlv, col])
                        plsc.store_scatter(
                            acc_v, [dlv, col],
                            old + rows_v[j, pl.ds(vv * 16, 16)])
                return cr

            lax.fori_loop(0, _BATCH, row, 0)
        return cz

    lax.fori_loop(0, _QCAP // _BATCH, batch, 0)

    @pl.when(wid < 2)
    def _():
        pltpu.sync_copy(acc_v.at[pl.ds(0, 320)], out_hbm.at[pl.ds(lo, 320)])

    @pl.when(wid >= 2)
    def _():
        pltpu.sync_copy(acc_v.at[pl.ds(0, 312)], out_hbm.at[pl.ds(lo, 312)])


@functools.partial(
    pl.kernel,
    out_type=jax.ShapeDtypeStruct((_N, 128), jnp.float32),
    mesh=plsc.VectorSubcoreMesh(core_axis_name="c", subcore_axis_name="s"),
    compiler_params=pltpu.CompilerParams(needs_layout_passes=False),
    scratch_types=[
        pltpu.VMEM((_CHUNK,), jnp.int32),
        pltpu.VMEM((_QCAP,), jnp.int32),
        pltpu.VMEM((_QCAP,), jnp.int32),
        pltpu.VMEM((_BATCH, 128), jnp.float32),
        pltpu.VMEM((320, 128), jnp.float32),
        pltpu.SemaphoreType.DMA,
    ],
)
def _seg_rowsum_kernel(upd_hbm, dst_hbm, out_hbm, dstc_v, idq_v, dlq_v,
                       rows_v, acc_v, sem):
    _rowsum_body(upd_hbm, dst_hbm, out_hbm, dstc_v, idq_v, dlq_v, rows_v,
                 acc_v, sem)


def _mm_body(x_ref, w_ref, as_ref, ad_ref, h_ref, als_ref, ald_ref):
    h = jnp.dot(x_ref[...], w_ref[...], preferred_element_type=jnp.float32)
    h_ref[...] = h
    als_ref[...] = jnp.dot(h, as_ref[...], preferred_element_type=jnp.float32)
    ald_ref[...] = jnp.dot(h, ad_ref[...], preferred_element_type=jnp.float32)


def _pallas_pre(x, w, a_src, a_dst):
    n = x.shape[0]
    h, als, ald = pl.pallas_call(
        _mm_body,
        out_shape=[
            jax.ShapeDtypeStruct((n, w.shape[1]), jnp.float32),
            jax.ShapeDtypeStruct((n, 1), jnp.float32),
            jax.ShapeDtypeStruct((n, 1), jnp.float32),
        ],
    )(x, w, a_src.reshape(-1, 1), a_dst.reshape(-1, 1))
    return h, als.reshape(-1), ald.reshape(-1)


def _gat_conv(x, edge_index, W, a_src, a_dst, b):
    h, al_src_p, al_dst_p = _pallas_pre(x, W, a_src, a_dst)
    src = edge_index[0]
    dst = edge_index[1]
    loop = jnp.arange(_N, dtype=src.dtype)
    src = jnp.concatenate([src, loop])
    dst = jnp.concatenate([dst, loop])
    e = _edge_score_kernel(al_src_p, al_dst_p, src, dst)
    e_max = jax.ops.segment_max(e, dst, num_segments=_N)
    e = _elemwise_1d(jnp.exp, _edge_sub_max_kernel(e_max, dst, e))
    denom = jax.ops.segment_sum(e, dst, num_segments=_N)
    alpha = _elemwise_1d(lambda t, d: t / d, e, _edge_take_kernel(denom, dst))
    upd = _row_gather_kernel(h, src, alpha)
    out = _seg_rowsum_kernel(upd, dst)
    return out + b


def _batch_norm(x, g, b):
    m = jnp.mean(x, axis=0)
    v = jnp.var(x, axis=0)
    return g * (x - m) / jnp.sqrt(v + _EPS) + b


def _elemwise_1d(fn, *arrays):
    n = arrays[0].shape[0]
    rows = -(-n // 128)
    rows = -(-rows // 8) * 8
    np_ = rows * 128

    def body(*refs):
        out_ref = refs[-1]
        out_ref[...] = fn(*[r[...] for r in refs[:-1]])

    padded = [jnp.pad(a, (0, np_ - n)).reshape(rows, 128) for a in arrays]
    out = pl.pallas_call(
        body,
        out_shape=jax.ShapeDtypeStruct((rows, 128), jnp.float32),
    )(*padded)
    return out.reshape(-1)[:n]


def kernel(x, edge_index, W1, a_src1, a_dst1, b1, g1, be1, W2, a_src2, a_dst2, b2, g2, be2, fc_w, fc_b):
    h = _gat_conv(x, edge_index, W1, a_src1, a_dst1, b1)
    h = _batch_norm(h, g1, be1)
    h = jax.nn.relu(h)
    h = _gat_conv(h, edge_index, W2, a_src2, a_dst2, b2)
    m = jnp.mean(h, axis=0)
    v = jnp.var(h, axis=0)
    h = g2 * (h - m) / jnp.sqrt(v + _EPS) + be2
    pooled = jnp.mean(h, axis=0, keepdims=True)
    out = pooled @ fc_w + fc_b
    return out.reshape(-1)
